# Initial kernel scaffold; baseline (speedup 1.0000x reference)
import jax
import jax.numpy as jnp
from jax.experimental import pallas as pl


def kernel(inputs_row, inputs_col, global_interaction, local_variation):
    raise NotImplementedError("write your pallas kernel here")


# speedup vs baseline: 2.3183x; 2.3183x over previous
import jax
import jax.numpy as jnp
from jax.experimental import pallas as pl
from jax.experimental.pallas import tpu as pltpu

_BLOCK = 4096


def _dedicom_body(row_ref, col_ref, g_ref, lv_ref, out_ref):
    row = row_ref[...]
    col = col_ref[...]
    g = g_ref[...]
    k_rel = lv_ref.shape[0]
    recs = []
    for k in range(k_rel):
        dk = lv_ref[k, :]
        left = jnp.dot(row * dk[None, :], g,
                       preferred_element_type=jnp.float32)
        recs.append(jnp.sum(left * (col * dk[None, :]), axis=1))
    scores = jnp.stack(recs, axis=0)
    out_ref[...] = jax.nn.sigmoid(scores)


def kernel(inputs_row, inputs_col, global_interaction, local_variation):
    n, d = inputs_row.shape
    k_rel = local_variation.shape[0]
    grid = (pl.cdiv(n, _BLOCK),)
    return pl.pallas_call(
        _dedicom_body,
        grid=grid,
        in_specs=[
            pl.BlockSpec((_BLOCK, d), lambda i: (i, 0)),
            pl.BlockSpec((_BLOCK, d), lambda i: (i, 0)),
            pl.BlockSpec((d, d), lambda i: (0, 0)),
            pl.BlockSpec((k_rel, d), lambda i: (0, 0)),
        ],
        out_specs=pl.BlockSpec((k_rel, _BLOCK), lambda i: (0, i)),
        out_shape=jax.ShapeDtypeStruct((k_rel, n), jnp.float32),
        compiler_params=pltpu.CompilerParams(
            dimension_semantics=("parallel",),
        ),
        name="dedicom_decoder",
    )(inputs_row, inputs_col, global_interaction, local_variation)
